# Initial kernel scaffold; baseline (speedup 1.0000x reference)
import jax
import jax.numpy as jnp
from jax.experimental import pallas as pl


def kernel(offset, points):
    raise NotImplementedError("write your pallas kernel here")


# speedup vs baseline: 25.9338x; 25.9338x over previous
import functools

import numpy as np
import jax
import jax.numpy as jnp
from jax import lax
from jax.experimental import pallas as pl
from jax.experimental.pallas import tpu as pltpu
from jax.experimental.pallas import tpu_sc as plsc

_T = 48
_LANES = 16
_GRP = 8
_CHUNK = _LANES * _GRP
_NC, _NS = 2, 16
_NW = _NC * _NS
_NM = 5
_NMP = 8


def _topo_w():
    w = np.sin(np.arange(_T * 8, dtype=np.float64).reshape(_T, 8))
    w = np.exp(w) / np.exp(w).sum(axis=1, keepdims=True)
    return w.astype(np.float32)


_CORNERS = np.array(
    [[di, dj, dk] for di in (0, 1) for dj in (0, 1) for dk in (0, 1)],
    dtype=np.float32)


def _sc_moments(xs, ys, zs, P, C):
    ncells = C * C * C
    per_tile = xs.shape[0] // _NW
    n_chunks = per_tile // _CHUNK
    rpt = ncells // _NS

    mesh = plsc.VectorSubcoreMesh(core_axis_name="c", subcore_axis_name="s")

    def body(xs_hbm, ys_hbm, zs_hbm, zeros_hbm, out_hbm,
             xb, yb, zb, cb0, cb1, cb2, cb3, cb4, idxb, zbuf,
             t0, t1, t2, t3, t4):
        tables = (t0, t1, t2, t3, t4)
        cbufs = (cb0, cb1, cb2, cb3, cb4)
        c = lax.axis_index("c")
        s = lax.axis_index("s")
        base = (c * _NS + s) * per_tile
        pltpu.sync_copy(xs_hbm.at[pl.ds(base, per_tile)], xb)
        pltpu.sync_copy(ys_hbm.at[pl.ds(base, per_tile)], yb)
        pltpu.sync_copy(zs_hbm.at[pl.ds(base, per_tile)], zb)
        for t in tables:
            pltpu.sync_copy(zeros_hbm, t.at[pl.ds(s * rpt, rpt)])
        pltpu.sync_copy(zeros_hbm, zbuf)
        plsc.subcore_barrier()

        lanes = lax.iota(jnp.int32, _LANES)

        def chunk_body(ch, carry):
            for g in range(_GRP):
                off = ch * _CHUNK + g * _LANES
                x = xb[pl.ds(off, _LANES)]
                y = yb[pl.ds(off, _LANES)]
                z = zb[pl.ds(off, _LANES)]
                xi = jnp.minimum(x.astype(jnp.int32), C - 1)
                yi = jnp.minimum(y.astype(jnp.int32), C - 1)
                zi = jnp.minimum(z.astype(jnp.int32), C - 1)
                px = x - xi.astype(jnp.float32)
                py = y - yi.astype(jnp.float32)
                pz = z - zi.astype(jnp.float32)
                q = px * px + py * py + pz * pz
                cid = (xi * C + yi) * C + zi
                gi = base + off + lanes
                cnt = jnp.where(gi < P, jnp.float32(1.0), jnp.float32(0.0))
                sl = pl.ds(g * _LANES, _LANES)
                cb0[sl] = cnt
                cb1[sl] = px
                cb2[sl] = py
                cb3[sl] = pz
                cb4[sl] = q
                idxb[sl] = cid
            for t, cb in zip(tables, cbufs):
                pltpu.sync_copy(cb, t.at[idxb], add=True)
            return carry

        lax.fori_loop(0, n_chunks, chunk_body, 0)
        plsc.subcore_barrier()
        for col, t in enumerate(tables):
            pltpu.sync_copy(
                t.at[pl.ds(s * rpt, rpt)],
                out_hbm.at[pl.ds((c * _NMP + col) * ncells + s * rpt, rpt)])
        for col in range(_NM, _NMP):
            pltpu.sync_copy(
                zbuf,
                out_hbm.at[pl.ds((c * _NMP + col) * ncells + s * rpt, rpt)])

    kfn = pl.kernel(
        body,
        out_type=jax.ShapeDtypeStruct((_NC * _NMP * ncells,), jnp.float32),
        mesh=mesh,
        scratch_types=[
            pltpu.VMEM((per_tile,), jnp.float32),
            pltpu.VMEM((per_tile,), jnp.float32),
            pltpu.VMEM((per_tile,), jnp.float32),
            pltpu.VMEM((_CHUNK,), jnp.float32),
            pltpu.VMEM((_CHUNK,), jnp.float32),
            pltpu.VMEM((_CHUNK,), jnp.float32),
            pltpu.VMEM((_CHUNK,), jnp.float32),
            pltpu.VMEM((_CHUNK,), jnp.float32),
            pltpu.VMEM((_CHUNK,), jnp.int32),
            pltpu.VMEM((rpt,), jnp.float32),
            pltpu.VMEM_SHARED((ncells,), jnp.float32),
            pltpu.VMEM_SHARED((ncells,), jnp.float32),
            pltpu.VMEM_SHARED((ncells,), jnp.float32),
            pltpu.VMEM_SHARED((ncells,), jnp.float32),
            pltpu.VMEM_SHARED((ncells,), jnp.float32),
        ],
    )
    return kfn(xs, ys, zs, jnp.zeros((rpt,), jnp.float32))


def _tc_body(C, x_ref, m_ref, wt_ref, cr_ref, sel_ref, o_ref):
    cdim = (((0,), (0,)), ((), ()))
    f32 = jnp.float32
    hp = lax.Precision.HIGHEST
    blk = x_ref.shape[2]
    i = pl.program_id(0)
    cid = i * blk + lax.broadcasted_iota(jnp.int32, (1, blk), 1)
    cellf = [(cid // (C * C)).astype(f32),
             ((cid // C) % C).astype(f32),
             (cid % C).astype(f32)]
    m = m_ref[0] + m_ref[1]
    mext = jnp.concatenate([m[0:5], cellf[0], cellf[1], cellf[2]], axis=0)
    mt = lax.dot_general(mext, sel_ref[...], cdim, precision=hp,
                         preferred_element_type=f32)
    cnt = mt[:, 0:1]
    sx = mt[:, 1:2]
    sy = mt[:, 2:3]
    sz = mt[:, 3:4]
    q = mt[:, 4:5]
    wtb = wt_ref[...].astype(jnp.bfloat16)
    a = []
    for d in range(3):
        base = cr_ref[:, d:d + 1] + cellf[d]
        xab = (base + x_ref[d]).astype(jnp.bfloat16)
        a.append(lax.dot_general(xab, wtb, cdim,
                                 preferred_element_type=f32))
    ax = a[0] - mt[:, 5:6]
    ay = a[1] - mt[:, 6:7]
    az = a[2] - mt[:, 7:8]
    n2 = ax * ax + ay * ay + az * az
    o_ref[...] = q + cnt * n2 - 2.0 * (sx * ax + sy * ay + sz * az)


def kernel(offset, points):
    N = offset.shape[1]
    C = N - 1
    ncells = C * C * C
    P = points.shape[0]

    per_tile = -(-P // (_NW * _CHUNK)) * _CHUNK
    ppad = per_tile * _NW
    pts_t = jnp.pad(jnp.transpose(points), ((0, 0), (0, ppad - P)))
    mflat = _sc_moments(pts_t[0], pts_t[1], pts_t[2], P, C)
    moments = mflat.reshape(_NC, _NMP, ncells)

    slabs = []
    for di in (0, 1):
        for dj in (0, 1):
            for dk in (0, 1):
                slabs.append(
                    offset[:, di:di + C, dj:dj + C, dk:dk + C].reshape(3, ncells))
    x_slabs = jnp.stack(slabs, axis=1)

    w = _topo_w()
    wt = jnp.asarray(w.T)
    cr = jnp.asarray(_CORNERS)
    sel = jnp.eye(_NMP, dtype=jnp.float32)

    blk = 4096
    out = pl.pallas_call(
        functools.partial(_tc_body, C),
        grid=(ncells // blk,),
        in_specs=[
            pl.BlockSpec((3, 8, blk), lambda i: (0, 0, i)),
            pl.BlockSpec((_NC, _NMP, blk), lambda i: (0, 0, i)),
            pl.BlockSpec((8, _T), lambda i: (0, 0)),
            pl.BlockSpec((8, 3), lambda i: (0, 0)),
            pl.BlockSpec((_NMP, _NMP), lambda i: (0, 0)),
        ],
        out_specs=pl.BlockSpec((blk, _T), lambda i: (i, 0)),
        out_shape=jax.ShapeDtypeStruct((ncells, _T), jnp.float32),
    )(x_slabs, moments, wt, cr, sel)
    return out


# speedup vs baseline: 46.4597x; 1.7915x over previous
import functools

import numpy as np
import jax
import jax.numpy as jnp
from jax import lax
from jax.experimental import pallas as pl
from jax.experimental.pallas import tpu as pltpu
from jax.experimental.pallas import tpu_sc as plsc

_T = 48
_LANES = 16
_GRP = 8
_CHUNK = _LANES * _GRP
_NC, _NS = 2, 16
_NW = _NC * _NS
_NM = 5
_NMP = 8


def _topo_w():
    w = np.sin(np.arange(_T * 8, dtype=np.float64).reshape(_T, 8))
    w = np.exp(w) / np.exp(w).sum(axis=1, keepdims=True)
    return w.astype(np.float32)


_CORNERS = np.array(
    [[di, dj, dk] for di in (0, 1) for dj in (0, 1) for dk in (0, 1)],
    dtype=np.float32)


def _sc_moments(xs, ys, zs, P, C):
    ncells = C * C * C
    per_tile = xs.shape[0] // _NW
    n_chunks = per_tile // _CHUNK
    rpt = ncells // _NS

    mesh = plsc.VectorSubcoreMesh(core_axis_name="c", subcore_axis_name="s")

    def body(xs_hbm, ys_hbm, zs_hbm, zeros_hbm, out_hbm,
             xb, yb, zb, cb0, cb1, cb2, cb3, cb4, idxb, zbuf,
             t0, t1, t2, t3, t4):
        tables = (t0, t1, t2, t3, t4)
        cbufs = (cb0, cb1, cb2, cb3, cb4)
        c = lax.axis_index("c")
        s = lax.axis_index("s")
        base = (c * _NS + s) * per_tile
        pltpu.sync_copy(xs_hbm.at[pl.ds(base, per_tile)], xb)
        pltpu.sync_copy(ys_hbm.at[pl.ds(base, per_tile)], yb)
        pltpu.sync_copy(zs_hbm.at[pl.ds(base, per_tile)], zb)
        for t in tables:
            pltpu.sync_copy(zeros_hbm, t.at[pl.ds(s * rpt, rpt)])
        pltpu.sync_copy(zeros_hbm, zbuf)
        plsc.subcore_barrier()

        lanes = lax.iota(jnp.int32, _LANES)

        def chunk_body(ch, carry):
            for g in range(_GRP):
                off = ch * _CHUNK + g * _LANES
                x = xb[pl.ds(off, _LANES)]
                y = yb[pl.ds(off, _LANES)]
                z = zb[pl.ds(off, _LANES)]
                xi = jnp.minimum(x.astype(jnp.int32), C - 1)
                yi = jnp.minimum(y.astype(jnp.int32), C - 1)
                zi = jnp.minimum(z.astype(jnp.int32), C - 1)
                px = x - xi.astype(jnp.float32)
                py = y - yi.astype(jnp.float32)
                pz = z - zi.astype(jnp.float32)
                q = px * px + py * py + pz * pz
                cid = (xi * C + yi) * C + zi
                gi = base + off + lanes
                cnt = jnp.where(gi < P, jnp.float32(1.0), jnp.float32(0.0))
                sl = pl.ds(g * _LANES, _LANES)
                cb0[sl] = cnt
                cb1[sl] = px
                cb2[sl] = py
                cb3[sl] = pz
                cb4[sl] = q
                idxb[sl] = cid
            for t, cb in zip(tables, cbufs):
                pltpu.sync_copy(cb, t.at[idxb], add=True)
            return carry

        lax.fori_loop(0, n_chunks, chunk_body, 0)
        plsc.subcore_barrier()
        for col, t in enumerate(tables):
            pltpu.sync_copy(
                t.at[pl.ds(s * rpt, rpt)],
                out_hbm.at[pl.ds((c * _NMP + col) * ncells + s * rpt, rpt)])
        for col in range(_NM, _NMP):
            pltpu.sync_copy(
                zbuf,
                out_hbm.at[pl.ds((c * _NMP + col) * ncells + s * rpt, rpt)])

    kfn = pl.kernel(
        body,
        out_type=jax.ShapeDtypeStruct((_NC * _NMP * ncells,), jnp.float32),
        mesh=mesh,
        scratch_types=[
            pltpu.VMEM((per_tile,), jnp.float32),
            pltpu.VMEM((per_tile,), jnp.float32),
            pltpu.VMEM((per_tile,), jnp.float32),
            pltpu.VMEM((_CHUNK,), jnp.float32),
            pltpu.VMEM((_CHUNK,), jnp.float32),
            pltpu.VMEM((_CHUNK,), jnp.float32),
            pltpu.VMEM((_CHUNK,), jnp.float32),
            pltpu.VMEM((_CHUNK,), jnp.float32),
            pltpu.VMEM((_CHUNK,), jnp.int32),
            pltpu.VMEM((rpt,), jnp.float32),
            pltpu.VMEM_SHARED((ncells,), jnp.float32),
            pltpu.VMEM_SHARED((ncells,), jnp.float32),
            pltpu.VMEM_SHARED((ncells,), jnp.float32),
            pltpu.VMEM_SHARED((ncells,), jnp.float32),
            pltpu.VMEM_SHARED((ncells,), jnp.float32),
        ],
    )
    return kfn(xs, ys, zs, jnp.zeros((rpt,), jnp.float32))


def _tc_body(C, x_ref, m_ref, wt_ref, cr_ref, o_ref):
    f32 = jnp.float32
    blk = x_ref.shape[2]
    i = pl.program_id(0)
    cid = i * blk + lax.broadcasted_iota(jnp.int32, (1, blk), 1)
    cellf = [(cid // (C * C)).astype(f32),
             ((cid // C) % C).astype(f32),
             (cid % C).astype(f32)]
    m = m_ref[0] + m_ref[1]
    cnt = m[0:1, :]
    sx = m[1:2, :]
    sy = m[2:3, :]
    sz = m[3:4, :]
    q = m[4:5, :]
    wb = wt_ref[...].astype(jnp.bfloat16)
    mm = (((1,), (0,)), ((), ()))
    a = []
    for d in range(3):
        base = cr_ref[:, d:d + 1] + cellf[d]
        xab = (base + x_ref[d]).astype(jnp.bfloat16)
        a.append(lax.dot_general(wb, xab, mm,
                                 preferred_element_type=f32))
    ax = a[0] - cellf[0]
    ay = a[1] - cellf[1]
    az = a[2] - cellf[2]
    n2 = ax * ax + ay * ay + az * az
    ot = q + cnt * n2 - 2.0 * (sx * ax + sy * ay + sz * az)
    o_ref[...] = jnp.swapaxes(ot, 0, 1)


def kernel(offset, points):
    N = offset.shape[1]
    C = N - 1
    ncells = C * C * C
    P = points.shape[0]

    per_tile = -(-P // (_NW * _CHUNK)) * _CHUNK
    ppad = per_tile * _NW
    pts_t = jnp.pad(jnp.transpose(points), ((0, 0), (0, ppad - P)))
    mflat = _sc_moments(pts_t[0], pts_t[1], pts_t[2], P, C)
    moments = mflat.reshape(_NC, _NMP, ncells)

    slabs = []
    for di in (0, 1):
        for dj in (0, 1):
            for dk in (0, 1):
                slabs.append(
                    offset[:, di:di + C, dj:dj + C, dk:dk + C].reshape(3, ncells))
    x_slabs = jnp.stack(slabs, axis=1)

    w = _topo_w()
    wt = jnp.asarray(w)
    cr = jnp.asarray(_CORNERS)

    blk = 4096
    out = pl.pallas_call(
        functools.partial(_tc_body, C),
        grid=(ncells // blk,),
        in_specs=[
            pl.BlockSpec((3, 8, blk), lambda i: (0, 0, i)),
            pl.BlockSpec((_NC, _NMP, blk), lambda i: (0, 0, i)),
            pl.BlockSpec((_T, 8), lambda i: (0, 0)),
            pl.BlockSpec((8, 3), lambda i: (0, 0)),
        ],
        out_specs=pl.BlockSpec((blk, _T), lambda i: (i, 0)),
        out_shape=jax.ShapeDtypeStruct((ncells, _T), jnp.float32),
    )(x_slabs, moments, wt, cr)
    return out


# speedup vs baseline: 46.4939x; 1.0007x over previous
import functools

import numpy as np
import jax
import jax.numpy as jnp
from jax import lax
from jax.experimental import pallas as pl
from jax.experimental.pallas import tpu as pltpu
from jax.experimental.pallas import tpu_sc as plsc

_T = 48
_LANES = 16
_GRP = 8
_CHUNK = _LANES * _GRP
_NC, _NS = 2, 16
_NW = _NC * _NS
_NM = 5
_NMP = 8


def _topo_w():
    w = np.sin(np.arange(_T * 8, dtype=np.float64).reshape(_T, 8))
    w = np.exp(w) / np.exp(w).sum(axis=1, keepdims=True)
    return w.astype(np.float32)


_CORNERS = np.array(
    [[di, dj, dk] for di in (0, 1) for dj in (0, 1) for dk in (0, 1)],
    dtype=np.float32)


def _sc_moments(xs, ys, zs, P, C):
    ncells = C * C * C
    per_tile = xs.shape[0] // _NW
    n_chunks = per_tile // _CHUNK
    rpt = ncells // _NS

    mesh = plsc.VectorSubcoreMesh(core_axis_name="c", subcore_axis_name="s")

    def body(xs_hbm, ys_hbm, zs_hbm, zeros_hbm, out_hbm,
             xb, yb, zb, cb0, cb1, cb2, cb3, cb4, idxb, zbuf,
             t0, t1, t2, t3, t4, sem):
        tables = (t0, t1, t2, t3, t4)
        cbufs = (cb0, cb1, cb2, cb3, cb4)
        c = lax.axis_index("c")
        s = lax.axis_index("s")
        base = (c * _NS + s) * per_tile
        pltpu.sync_copy(xs_hbm.at[pl.ds(base, per_tile)], xb)
        pltpu.sync_copy(ys_hbm.at[pl.ds(base, per_tile)], yb)
        pltpu.sync_copy(zs_hbm.at[pl.ds(base, per_tile)], zb)
        for t in tables:
            pltpu.sync_copy(zeros_hbm, t.at[pl.ds(s * rpt, rpt)])
        pltpu.sync_copy(zeros_hbm, zbuf)
        plsc.subcore_barrier()

        lanes = lax.iota(jnp.int32, _LANES)

        def chunk_body(ch, carry):
            for g in range(_GRP):
                off = ch * _CHUNK + g * _LANES
                x = xb[pl.ds(off, _LANES)]
                y = yb[pl.ds(off, _LANES)]
                z = zb[pl.ds(off, _LANES)]
                xi = jnp.minimum(x.astype(jnp.int32), C - 1)
                yi = jnp.minimum(y.astype(jnp.int32), C - 1)
                zi = jnp.minimum(z.astype(jnp.int32), C - 1)
                px = x - xi.astype(jnp.float32)
                py = y - yi.astype(jnp.float32)
                pz = z - zi.astype(jnp.float32)
                q = px * px + py * py + pz * pz
                cid = (xi * C + yi) * C + zi
                gi = base + off + lanes
                cnt = jnp.where(gi < P, jnp.float32(1.0), jnp.float32(0.0))
                sl = pl.ds(g * _LANES, _LANES)
                cb0[sl] = cnt
                cb1[sl] = px
                cb2[sl] = py
                cb3[sl] = pz
                cb4[sl] = q
                idxb[sl] = cid
            handles = [pltpu.async_copy(cb, t.at[idxb], sem, add=True)
                       for t, cb in zip(tables, cbufs)]
            for h in handles:
                h.wait()
            return carry

        lax.fori_loop(0, n_chunks, chunk_body, 0)
        plsc.subcore_barrier()
        for col, t in enumerate(tables):
            pltpu.sync_copy(
                t.at[pl.ds(s * rpt, rpt)],
                out_hbm.at[pl.ds((c * _NMP + col) * ncells + s * rpt, rpt)])
        for col in range(_NM, _NMP):
            pltpu.sync_copy(
                zbuf,
                out_hbm.at[pl.ds((c * _NMP + col) * ncells + s * rpt, rpt)])

    kfn = pl.kernel(
        body,
        out_type=jax.ShapeDtypeStruct((_NC * _NMP * ncells,), jnp.float32),
        mesh=mesh,
        scratch_types=[
            pltpu.VMEM((per_tile,), jnp.float32),
            pltpu.VMEM((per_tile,), jnp.float32),
            pltpu.VMEM((per_tile,), jnp.float32),
            pltpu.VMEM((_CHUNK,), jnp.float32),
            pltpu.VMEM((_CHUNK,), jnp.float32),
            pltpu.VMEM((_CHUNK,), jnp.float32),
            pltpu.VMEM((_CHUNK,), jnp.float32),
            pltpu.VMEM((_CHUNK,), jnp.float32),
            pltpu.VMEM((_CHUNK,), jnp.int32),
            pltpu.VMEM((rpt,), jnp.float32),
            pltpu.VMEM_SHARED((ncells,), jnp.float32),
            pltpu.VMEM_SHARED((ncells,), jnp.float32),
            pltpu.VMEM_SHARED((ncells,), jnp.float32),
            pltpu.VMEM_SHARED((ncells,), jnp.float32),
            pltpu.VMEM_SHARED((ncells,), jnp.float32),
            pltpu.SemaphoreType.DMA,
        ],
    )
    return kfn(xs, ys, zs, jnp.zeros((rpt,), jnp.float32))


def _tc_body(C, x_ref, m_ref, wt_ref, cr_ref, o_ref):
    f32 = jnp.float32
    blk = x_ref.shape[2]
    i = pl.program_id(0)
    cid = i * blk + lax.broadcasted_iota(jnp.int32, (1, blk), 1)
    cellf = [(cid // (C * C)).astype(f32),
             ((cid // C) % C).astype(f32),
             (cid % C).astype(f32)]
    m = m_ref[0] + m_ref[1]
    cnt = m[0:1, :]
    sx = m[1:2, :]
    sy = m[2:3, :]
    sz = m[3:4, :]
    q = m[4:5, :]
    wb = wt_ref[...].astype(jnp.bfloat16)
    mm = (((1,), (0,)), ((), ()))
    a = []
    for d in range(3):
        base = cr_ref[:, d:d + 1] + cellf[d]
        xab = (base + x_ref[d]).astype(jnp.bfloat16)
        a.append(lax.dot_general(wb, xab, mm,
                                 preferred_element_type=f32))
    ax = a[0] - cellf[0]
    ay = a[1] - cellf[1]
    az = a[2] - cellf[2]
    n2 = ax * ax + ay * ay + az * az
    ot = q + cnt * n2 - 2.0 * (sx * ax + sy * ay + sz * az)
    o_ref[...] = jnp.swapaxes(ot, 0, 1)


def kernel(offset, points):
    N = offset.shape[1]
    C = N - 1
    ncells = C * C * C
    P = points.shape[0]

    per_tile = -(-P // (_NW * _CHUNK)) * _CHUNK
    ppad = per_tile * _NW
    pts_t = jnp.pad(jnp.transpose(points), ((0, 0), (0, ppad - P)))
    mflat = _sc_moments(pts_t[0], pts_t[1], pts_t[2], P, C)
    moments = mflat.reshape(_NC, _NMP, ncells)

    slabs = []
    for di in (0, 1):
        for dj in (0, 1):
            for dk in (0, 1):
                slabs.append(
                    offset[:, di:di + C, dj:dj + C, dk:dk + C].reshape(3, ncells))
    x_slabs = jnp.stack(slabs, axis=1)

    w = _topo_w()
    wt = jnp.asarray(w)
    cr = jnp.asarray(_CORNERS)

    blk = 4096
    out = pl.pallas_call(
        functools.partial(_tc_body, C),
        grid=(ncells // blk,),
        in_specs=[
            pl.BlockSpec((3, 8, blk), lambda i: (0, 0, i)),
            pl.BlockSpec((_NC, _NMP, blk), lambda i: (0, 0, i)),
            pl.BlockSpec((_T, 8), lambda i: (0, 0)),
            pl.BlockSpec((8, 3), lambda i: (0, 0)),
        ],
        out_specs=pl.BlockSpec((blk, _T), lambda i: (i, 0)),
        out_shape=jax.ShapeDtypeStruct((ncells, _T), jnp.float32),
    )(x_slabs, moments, wt, cr)
    return out
